# Initial kernel scaffold; baseline (speedup 1.0000x reference)
import jax
import jax.numpy as jnp
from jax.experimental import pallas as pl


def kernel(x, in_degree, out_degree, atom_table, in_degree_table, out_degree_table, graph_token, token_init):
    raise NotImplementedError("write your pallas kernel here")


# speedup vs baseline: 2.2133x; 2.2133x over previous
import jax
import jax.numpy as jnp
from jax import lax
from jax.experimental import pallas as pl
from jax.experimental.pallas import tpu as pltpu
from jax.experimental.pallas import tpu_sc as plsc

B, N, F, H = 256, 128, 9, 768
L = 16
G = 8
CHUNKS = N // G
NC, NS = 2, 16
NW = NC * NS
GPW = B // NW
NP1 = N + 1


def _sc_body(x_hbm, ind_hbm, outd_hbm, atom_hbm, int_hbm, outt_hbm, gt_hbm,
             out_hbm,
             xidx_v, iidx_v, oidx_v, xrows_v, irows_v, orows_v, obuf_v, gt_v,
             sem_x, sem_i, sem_o):
    wid = lax.axis_index("s") * NC + lax.axis_index("c")
    pltpu.sync_copy(gt_hbm, gt_v)

    def graph_body(gi, carry):
        b = wid * GPW + gi
        pltpu.sync_copy(gt_v, out_hbm.at[pl.ds(b * NP1 * H, H)])

        def chunk_body(ci, carry2):
            n0 = ci * G
            pltpu.sync_copy(x_hbm.at[pl.ds((b * N + n0) * F, G * F)], xidx_v)
            pltpu.sync_copy(ind_hbm.at[pl.ds(b * N + n0, G)], iidx_v)
            pltpu.sync_copy(outd_hbm.at[pl.ds(b * N + n0, G)], oidx_v)
            cx = pltpu.async_copy(atom_hbm.at[xidx_v], xrows_v, sem_x)
            cin = pltpu.async_copy(int_hbm.at[iidx_v], irows_v, sem_i)
            cout = pltpu.async_copy(outt_hbm.at[oidx_v], orows_v, sem_o)
            cx.wait()
            cin.wait()
            cout.wait()

            def col_body(j, carry3):
                off = j * L
                for g in range(G):
                    acc = irows_v[g, pl.ds(off, L)] + orows_v[g, pl.ds(off, L)]
                    for f in range(F):
                        acc = acc + xrows_v[g * F + f, pl.ds(off, L)]
                    obuf_v[pl.ds(g * H + off, L)] = acc
                return carry3

            lax.fori_loop(0, H // L, col_body, 0)
            pltpu.sync_copy(
                obuf_v, out_hbm.at[pl.ds((b * NP1 + 1 + n0) * H, G * H)])
            return carry2

        lax.fori_loop(0, CHUNKS, chunk_body, 0)
        return carry

    lax.fori_loop(0, GPW, graph_body, 0)


def kernel(x, in_degree, out_degree, atom_table, in_degree_table,
           out_degree_table, graph_token, token_init):
    del token_init
    x_flat = x.reshape(-1)
    ind_flat = in_degree.reshape(-1)
    outd_flat = out_degree.reshape(-1)
    gt_flat = graph_token.reshape(-1)
    mesh = plsc.VectorSubcoreMesh(core_axis_name="c", subcore_axis_name="s")
    k = pl.kernel(
        _sc_body,
        out_type=jax.ShapeDtypeStruct((B * NP1 * H,), jnp.float32),
        mesh=mesh,
        scratch_types=[
            pltpu.VMEM((G * F,), jnp.int32),
            pltpu.VMEM((G,), jnp.int32),
            pltpu.VMEM((G,), jnp.int32),
            pltpu.VMEM((G * F, H), jnp.float32),
            pltpu.VMEM((G, H), jnp.float32),
            pltpu.VMEM((G, H), jnp.float32),
            pltpu.VMEM((G * H,), jnp.float32),
            pltpu.VMEM((H,), jnp.float32),
            pltpu.SemaphoreType.DMA,
            pltpu.SemaphoreType.DMA,
            pltpu.SemaphoreType.DMA,
        ],
    )
    out_flat = k(x_flat, ind_flat, outd_flat, atom_table, in_degree_table,
                 out_degree_table, gt_flat)
    return out_flat.reshape(B, NP1, H)
